# baseline (speedup 1.0000x reference)
import jax
import jax.numpy as jnp
from jax.experimental import pallas as pl
from jax.experimental.pallas import tpu as pltpu
from jax.experimental.pallas import tpu_sc as plsc

_BM = 512
_BM2 = 2048
_GW = 128
_CK = 576


def _insert3(carry, bv, bi):
    a0, a1, a2, i0, i1, i2 = carry
    t = bv < a2
    v2 = jnp.where(t, bv, a2)
    j2 = jnp.where(t, bi, i2)
    s = v2 < a1
    nv1 = jnp.where(s, v2, a1)
    nj1 = jnp.where(s, j2, i1)
    nv2 = jnp.where(s, a1, v2)
    nj2 = jnp.where(s, i1, j2)
    s0 = nv1 < a0
    fv0 = jnp.where(s0, nv1, a0)
    fj0 = jnp.where(s0, nj1, i0)
    fv1 = jnp.where(s0, a0, nv1)
    fj1 = jnp.where(s0, i0, nj1)
    return fv0, fv1, nv2, fj0, fj1, nj2


def _knn_body(psT_ref, bsk_ref, pos_ref, b_ref, b_row_ref, idx_ref, nw_ref):
    n = pos_ref.shape[0]
    blo = bsk_ref[0, 0]
    bhi = bsk_ref[0, _BM - 1]
    b_row = b_row_ref[...]
    lo = jnp.sum((b_row < blo).astype(jnp.int32))
    hi = jnp.sum((b_row <= bhi).astype(jnp.int32))
    qx = psT_ref[0:1, :]
    qy = psT_ref[1:2, :]
    qz = psT_ref[2:3, :]
    bq = bsk_ref[...]
    iota = jax.lax.broadcasted_iota(jnp.int32, (_CK, _BM), 0).astype(jnp.float32)
    big = jnp.float32(1e10)
    lo8 = (lo // 8) * 8

    def chunk_step(t, carry):
        ubase = lo8 + t * _CK
        base = jnp.minimum(ubase, n - _CK)
        cx = pos_ref[pl.ds(base, _CK), 0:1]
        cy = pos_ref[pl.ds(base, _CK), 1:2]
        cz = pos_ref[pl.ds(base, _CK), 2:3]
        bc = b_ref[pl.ds(base, _CK), 0:1]
        dx = cx - qx
        dy = cy - qy
        dz = cz - qz
        d = dx * dx + dy * dy + dz * dz
        d = jnp.where(bc != bq, big, d)
        thresh = (ubase - base).astype(jnp.float32)
        d = jnp.where(iota < thresh, big, d)
        basef = base.astype(jnp.float32)
        for k in range(3):
            m = jnp.min(d, axis=0, keepdims=True)
            eq = d == m
            candf = jnp.where(eq, iota, jnp.float32(_CK))
            ikf = jnp.min(candf, axis=0, keepdims=True)
            if k < 2:
                d = jnp.where(eq, jnp.float32(3.4e38), d)
            carry = _insert3(carry, m, ikf + basef)
        return carry

    ones = jnp.ones((1, _BM), jnp.float32)
    init = (big * ones, big * ones, big * ones,
            0.0 * ones, 1.0 * ones, 2.0 * ones)
    trips = (hi - lo8 + _CK - 1) // _CK
    a0, a1, a2, i0, i1, i2 = jax.lax.fori_loop(0, trips, chunk_step, init)
    ws = [1.0 / jnp.maximum(v, jnp.float32(1e-16)) for v in (a0, a1, a2)]
    den = ws[0] + ws[1] + ws[2]
    idx_ref[...] = jnp.concatenate([i0, i1, i2], axis=0).astype(jnp.int32)
    nw_ref[...] = jnp.concatenate([w / den for w in ws], axis=0)


def _knn_topk(pos_skipT, bsk_row, pos, b_col, b_row):
    m = pos_skipT.shape[1]
    n = pos.shape[0]
    grid = (m // _BM,)
    return pl.pallas_call(
        _knn_body,
        grid=grid,
        in_specs=[
            pl.BlockSpec((3, _BM), lambda i: (0, i)),
            pl.BlockSpec((1, _BM), lambda i: (0, i)),
            pl.BlockSpec((n, 3), lambda i: (0, 0)),
            pl.BlockSpec((n, 1), lambda i: (0, 0)),
            pl.BlockSpec((1, n), lambda i: (0, 0)),
        ],
        out_specs=[
            pl.BlockSpec((3, _BM), lambda i: (0, i)),
            pl.BlockSpec((3, _BM), lambda i: (0, i)),
        ],
        out_shape=[
            jax.ShapeDtypeStruct((3, m), jnp.int32),
            jax.ShapeDtypeStruct((3, m), jnp.float32),
        ],
    )(pos_skipT, bsk_row, pos, b_col, b_row)


def _sc_gather(x, idx):
    r, m = idx.shape
    num_idx = r * m
    d = x.shape[1]
    npr = m // _GW
    mesh = plsc.VectorSubcoreMesh(core_axis_name="c", subcore_axis_name="s")

    @pl.kernel(
        out_type=jax.ShapeDtypeStruct((num_idx, d), x.dtype),
        mesh=mesh,
    )
    def kern(x_hbm, i_hbm, o_hbm):
        def body(i_vmem, o_vmem):
            pltpu.sync_copy(x_hbm.at[i_vmem.at[0]], o_vmem)

        pltpu.emit_pipeline(
            body,
            grid=(num_idx // _GW,),
            in_specs=[pl.BlockSpec((1, _GW),
                                   index_map=lambda i: (i // npr, i % npr))],
            out_specs=[pl.BlockSpec((_GW, d), index_map=lambda i: (i, 0))],
            core_axis_name=("c", "s"),
            dimension_semantics=(pltpu.PARALLEL,),
        )(i_hbm, o_hbm)

    return kern(x, idx)


def _mlp_body(g0_ref, g1_ref, g2_ref, nw_ref, xs_ref, w1a_ref, w1b_ref,
              b1_ref, w2_ref, b2_ref, out_ref):
    w0 = nw_ref[:, 0:1]
    w1 = nw_ref[:, 1:2]
    w2c = nw_ref[:, 2:3]
    h = g0_ref[...] * w0 + g1_ref[...] * w1 + g2_ref[...] * w2c
    f = jnp.float32
    z = (jnp.dot(h.astype(jnp.bfloat16), w1a_ref[...], preferred_element_type=f)
         + jnp.dot(xs_ref[...].astype(jnp.bfloat16), w1b_ref[...],
                   preferred_element_type=f)
         + b1_ref[...])
    z = jnp.maximum(z, 0.0)
    out_ref[...] = jnp.dot(z.astype(jnp.bfloat16), w2_ref[...],
                           preferred_element_type=f) + b2_ref[...]


def _mlp(gathered, nw, x_skip, w1a, w1b, b1r, w2, b2r):
    m = nw.shape[0]
    d_in = gathered.shape[1]
    d_skip = x_skip.shape[1]
    d_hid = w2.shape[0]
    d_out = w2.shape[1]
    nblk = m // _BM2
    return pl.pallas_call(
        _mlp_body,
        grid=(nblk,),
        in_specs=[
            pl.BlockSpec((_BM2, d_in), lambda i: (i, 0)),
            pl.BlockSpec((_BM2, d_in), lambda i: (i + nblk, 0)),
            pl.BlockSpec((_BM2, d_in), lambda i: (i + 2 * nblk, 0)),
            pl.BlockSpec((_BM2, 3), lambda i: (i, 0)),
            pl.BlockSpec((_BM2, d_skip), lambda i: (i, 0)),
            pl.BlockSpec((d_in, d_hid), lambda i: (0, 0)),
            pl.BlockSpec((d_skip, d_hid), lambda i: (0, 0)),
            pl.BlockSpec((1, d_hid), lambda i: (0, 0)),
            pl.BlockSpec((d_hid, d_out), lambda i: (0, 0)),
            pl.BlockSpec((1, d_out), lambda i: (0, 0)),
        ],
        out_specs=pl.BlockSpec((_BM2, d_out), lambda i: (i, 0)),
        out_shape=jax.ShapeDtypeStruct((m, d_out), jnp.float32),
    )(gathered, gathered, gathered, nw, x_skip,
      w1a.astype(jnp.bfloat16), w1b.astype(jnp.bfloat16), b1r,
      w2.astype(jnp.bfloat16), b2r)


def kernel(x, pos, batch, x_skip, pos_skip, batch_skip, W1, b1, W2, b2):
    n = x.shape[0]
    m = x_skip.shape[0]
    d_in = x.shape[1]

    b32 = batch.astype(jnp.int32)
    bs32 = batch_skip.astype(jnp.int32)
    b_col = b32.reshape(n, 1)
    b_row = b32.reshape(1, n)
    bsk_row = bs32.reshape(1, m)
    pos_skipT = pos_skip.T

    idx, nw = _knn_topk(pos_skipT, bsk_row, pos, b_col, b_row)

    gathered = _sc_gather(x, idx)

    w1a = W1[:d_in]
    w1b = W1[d_in:]
    return _mlp(gathered, nw.T, x_skip, w1a, w1b, b1.reshape(1, -1), W2,
                b2.reshape(1, -1))


# speedup vs baseline: 1.0070x; 1.0070x over previous
import jax
import jax.numpy as jnp
from jax.experimental import pallas as pl
from jax.experimental.pallas import tpu as pltpu
from jax.experimental.pallas import tpu_sc as plsc

_BM = 512
_BM2 = 2048
_GW = 128
_CK = 576


def _insert3(carry, bv, bi):
    a0, a1, a2, i0, i1, i2 = carry
    t = bv < a2
    v2 = jnp.where(t, bv, a2)
    j2 = jnp.where(t, bi, i2)
    s = v2 < a1
    nv1 = jnp.where(s, v2, a1)
    nj1 = jnp.where(s, j2, i1)
    nv2 = jnp.where(s, a1, v2)
    nj2 = jnp.where(s, i1, j2)
    s0 = nv1 < a0
    fv0 = jnp.where(s0, nv1, a0)
    fj0 = jnp.where(s0, nj1, i0)
    fv1 = jnp.where(s0, a0, nv1)
    fj1 = jnp.where(s0, i0, nj1)
    return fv0, fv1, nv2, fj0, fj1, nj2


def _knn_body(psT_ref, bsk_ref, pos_ref, b_ref, b_row_ref, idx_ref, nw_ref):
    n = pos_ref.shape[0]
    blo = bsk_ref[0, 0]
    bhi = bsk_ref[0, _BM - 1]
    b_row = b_row_ref[...]
    lo = jnp.sum((b_row < blo).astype(jnp.int32))
    hi = jnp.sum((b_row <= bhi).astype(jnp.int32))
    qx = psT_ref[0:1, :]
    qy = psT_ref[1:2, :]
    qz = psT_ref[2:3, :]
    bq = bsk_ref[...]
    iota = jax.lax.broadcasted_iota(jnp.int32, (_CK, _BM), 0).astype(jnp.float32)
    iota_col = jax.lax.broadcasted_iota(jnp.int32, (_CK, 1), 0)
    big = jnp.float32(1e10)
    lo8 = (lo // 8) * 8

    def chunk_step(t, carry):
        ubase = lo8 + t * _CK
        base = jnp.minimum(ubase, n - _CK)
        cx = pos_ref[pl.ds(base, _CK), 0:1]
        cy = pos_ref[pl.ds(base, _CK), 1:2]
        cz = pos_ref[pl.ds(base, _CK), 2:3]
        bc = b_ref[pl.ds(base, _CK), 0:1]
        bc = jnp.where(iota_col < (ubase - base), jnp.int32(-1), bc)
        dx = cx - qx
        dy = cy - qy
        dz = cz - qz
        d = dx * dx + dy * dy + dz * dz
        d = jnp.where(bc != bq, big, d)
        basef = base.astype(jnp.float32)
        for k in range(3):
            m = jnp.min(d, axis=0, keepdims=True)
            eq = d == m
            candf = jnp.where(eq, iota, jnp.float32(_CK))
            ikf = jnp.min(candf, axis=0, keepdims=True)
            if k < 2:
                d = jnp.where(eq, jnp.float32(3.4e38), d)
            carry = _insert3(carry, m, ikf + basef)
        return carry

    ones = jnp.ones((1, _BM), jnp.float32)
    init = (big * ones, big * ones, big * ones,
            0.0 * ones, 1.0 * ones, 2.0 * ones)
    trips = (hi - lo8 + _CK - 1) // _CK
    a0, a1, a2, i0, i1, i2 = jax.lax.fori_loop(0, trips, chunk_step, init)
    ws = [1.0 / jnp.maximum(v, jnp.float32(1e-16)) for v in (a0, a1, a2)]
    den = ws[0] + ws[1] + ws[2]
    idx_ref[...] = jnp.concatenate([i0, i1, i2], axis=0).astype(jnp.int32)
    nw_ref[...] = jnp.concatenate([w / den for w in ws], axis=0)


def _knn_topk(pos_skipT, bsk_row, pos, b_col, b_row):
    m = pos_skipT.shape[1]
    n = pos.shape[0]
    grid = (m // _BM,)
    return pl.pallas_call(
        _knn_body,
        grid=grid,
        in_specs=[
            pl.BlockSpec((3, _BM), lambda i: (0, i)),
            pl.BlockSpec((1, _BM), lambda i: (0, i)),
            pl.BlockSpec((n, 3), lambda i: (0, 0)),
            pl.BlockSpec((n, 1), lambda i: (0, 0)),
            pl.BlockSpec((1, n), lambda i: (0, 0)),
        ],
        out_specs=[
            pl.BlockSpec((3, _BM), lambda i: (0, i)),
            pl.BlockSpec((3, _BM), lambda i: (0, i)),
        ],
        out_shape=[
            jax.ShapeDtypeStruct((3, m), jnp.int32),
            jax.ShapeDtypeStruct((3, m), jnp.float32),
        ],
    )(pos_skipT, bsk_row, pos, b_col, b_row)


def _sc_gather(x, idx):
    r, m = idx.shape
    num_idx = r * m
    d = x.shape[1]
    npr = m // _GW
    mesh = plsc.VectorSubcoreMesh(core_axis_name="c", subcore_axis_name="s")

    @pl.kernel(
        out_type=jax.ShapeDtypeStruct((num_idx, d), x.dtype),
        mesh=mesh,
    )
    def kern(x_hbm, i_hbm, o_hbm):
        def body(i_vmem, o_vmem):
            pltpu.sync_copy(x_hbm.at[i_vmem.at[0]], o_vmem)

        pltpu.emit_pipeline(
            body,
            grid=(num_idx // _GW,),
            in_specs=[pl.BlockSpec((1, _GW),
                                   index_map=lambda i: (i // npr, i % npr))],
            out_specs=[pl.BlockSpec((_GW, d), index_map=lambda i: (i, 0))],
            core_axis_name=("c", "s"),
            dimension_semantics=(pltpu.PARALLEL,),
        )(i_hbm, o_hbm)

    return kern(x, idx)


def _mlp_body(g0_ref, g1_ref, g2_ref, nw_ref, xs_ref, w1a_ref, w1b_ref,
              b1_ref, w2_ref, b2_ref, out_ref):
    w0 = nw_ref[:, 0:1]
    w1 = nw_ref[:, 1:2]
    w2c = nw_ref[:, 2:3]
    f32 = jnp.float32
    h = (g0_ref[...].astype(f32) * w0 + g1_ref[...].astype(f32) * w1
         + g2_ref[...].astype(f32) * w2c)
    f = jnp.float32
    z = (jnp.dot(h.astype(jnp.bfloat16), w1a_ref[...], preferred_element_type=f)
         + jnp.dot(xs_ref[...].astype(jnp.bfloat16), w1b_ref[...],
                   preferred_element_type=f)
         + b1_ref[...])
    z = jnp.maximum(z, 0.0)
    out_ref[...] = jnp.dot(z.astype(jnp.bfloat16), w2_ref[...],
                           preferred_element_type=f) + b2_ref[...]


def _mlp(gathered, nw, x_skip, w1a, w1b, b1r, w2, b2r):
    m = nw.shape[0]
    d_in = gathered.shape[1]
    d_skip = x_skip.shape[1]
    d_hid = w2.shape[0]
    d_out = w2.shape[1]
    nblk = m // _BM2
    return pl.pallas_call(
        _mlp_body,
        grid=(nblk,),
        in_specs=[
            pl.BlockSpec((_BM2, d_in), lambda i: (i, 0)),
            pl.BlockSpec((_BM2, d_in), lambda i: (i + nblk, 0)),
            pl.BlockSpec((_BM2, d_in), lambda i: (i + 2 * nblk, 0)),
            pl.BlockSpec((_BM2, 3), lambda i: (i, 0)),
            pl.BlockSpec((_BM2, d_skip), lambda i: (i, 0)),
            pl.BlockSpec((d_in, d_hid), lambda i: (0, 0)),
            pl.BlockSpec((d_skip, d_hid), lambda i: (0, 0)),
            pl.BlockSpec((1, d_hid), lambda i: (0, 0)),
            pl.BlockSpec((d_hid, d_out), lambda i: (0, 0)),
            pl.BlockSpec((1, d_out), lambda i: (0, 0)),
        ],
        out_specs=pl.BlockSpec((_BM2, d_out), lambda i: (i, 0)),
        out_shape=jax.ShapeDtypeStruct((m, d_out), jnp.float32),
    )(gathered, gathered, gathered, nw, x_skip,
      w1a.astype(jnp.bfloat16), w1b.astype(jnp.bfloat16), b1r,
      w2.astype(jnp.bfloat16), b2r)


def kernel(x, pos, batch, x_skip, pos_skip, batch_skip, W1, b1, W2, b2):
    n = x.shape[0]
    m = x_skip.shape[0]
    d_in = x.shape[1]

    b32 = batch.astype(jnp.int32)
    bs32 = batch_skip.astype(jnp.int32)
    b_col = b32.reshape(n, 1)
    b_row = b32.reshape(1, n)
    bsk_row = bs32.reshape(1, m)
    pos_skipT = pos_skip.T

    idx, nw = _knn_topk(pos_skipT, bsk_row, pos, b_col, b_row)

    gathered = _sc_gather(x, idx)

    w1a = W1[:d_in]
    w1b = W1[d_in:]
    return _mlp(gathered, nw.T, x_skip, w1a, w1b, b1.reshape(1, -1), W2,
                b2.reshape(1, -1))
